# Initial kernel scaffold; baseline (speedup 1.0000x reference)
import jax
import jax.numpy as jnp
from jax.experimental import pallas as pl


def kernel(x, edge_index, W1, b1, W2, b2):
    raise NotImplementedError("write your pallas kernel here")


# speedup vs baseline: 24.9051x; 24.9051x over previous
import functools

import jax
import jax.numpy as jnp
from jax import lax
from jax.experimental import pallas as pl
from jax.experimental.pallas import tpu as pltpu
from jax.experimental.pallas import tpu_sc as plsc

N = 10000
D = 128
E = 320000

NC = 2
NS = 16
NW = NC * NS

CHUNK = 128
EPW = -(-E // (NW * CHUNK)) * CHUNK
E_PAD = EPW * NW
NCHUNK = EPW // CHUNK

ACC_ROWS = 10240
PAD_ROWS = ACC_ROWS - N
RPS = ACC_ROWS // NS

RB = 2000


_mesh = plsc.VectorSubcoreMesh(
    core_axis_name="c", subcore_axis_name="s", num_cores=NC, num_subcores=NS
)


@functools.partial(
    pl.kernel,
    out_type=jax.ShapeDtypeStruct((NC * ACC_ROWS,), jnp.float32),
    mesh=_mesh,
    scratch_types=[
        pltpu.VMEM((NCHUNK, CHUNK), jnp.int32),
        pltpu.VMEM((CHUNK,), jnp.float32),
        pltpu.VMEM((RPS,), jnp.float32),
        pltpu.VMEM_SHARED((ACC_ROWS,), jnp.float32),
    ],
)
def _deg_kernel(dst_hbm, out_hbm, dst_v, ones_v, zbuf, acc_sh):
    c = lax.axis_index("c")
    s = lax.axis_index("s")
    w = c * NS + s
    for k in range(CHUNK // 16):
        ones_v[pl.ds(k * 16, 16)] = jnp.ones((16,), jnp.float32)

    def zbody(i, carry):
        zbuf[pl.ds(i * 16, 16)] = jnp.zeros((16,), jnp.float32)
        return carry

    lax.fori_loop(0, RPS // 16, zbody, 0)
    pltpu.sync_copy(zbuf, acc_sh.at[pl.ds(s * RPS, RPS)])
    pltpu.sync_copy(dst_hbm.at[w], dst_v)
    plsc.subcore_barrier()

    def body(j, carry):
        pltpu.sync_copy(ones_v, acc_sh.at[dst_v.at[j]], add=True)
        return carry

    lax.fori_loop(0, NCHUNK, body, 0)
    plsc.subcore_barrier()
    pltpu.sync_copy(acc_sh.at[pl.ds(s * RPS, RPS)], zbuf)
    pltpu.sync_copy(zbuf, out_hbm.at[pl.ds(c * ACC_ROWS + s * RPS, RPS)])


@functools.partial(
    pl.kernel,
    out_type=jax.ShapeDtypeStruct((NC, ACC_ROWS, D), jnp.float32),
    mesh=_mesh,
    scratch_types=[
        pltpu.VMEM((NCHUNK, CHUNK), jnp.int32),
        pltpu.VMEM((NCHUNK, CHUNK), jnp.int32),
        pltpu.VMEM((CHUNK, D), jnp.float32),
        pltpu.VMEM_SHARED((ACC_ROWS, D), jnp.float32),
    ],
)
def _agg_kernel(table_hbm, src_hbm, dst_hbm, out_hbm,
                src_v, dst_v, gbuf, acc_sh):
    c = lax.axis_index("c")
    s = lax.axis_index("s")
    w = c * NS + s

    def zbody(i, carry):
        for k in range(D // 16):
            gbuf[i, pl.ds(k * 16, 16)] = jnp.zeros((16,), jnp.float32)
        return carry

    lax.fori_loop(0, CHUNK, zbody, 0)
    for r in range(RPS // CHUNK):
        pltpu.sync_copy(
            gbuf, acc_sh.at[pl.ds(s * RPS + r * CHUNK, CHUNK)]
        )
    pltpu.sync_copy(src_hbm.at[w], src_v)
    pltpu.sync_copy(dst_hbm.at[w], dst_v)
    plsc.subcore_barrier()

    def body(j, carry):
        pltpu.sync_copy(table_hbm.at[src_v.at[j]], gbuf)
        pltpu.sync_copy(gbuf, acc_sh.at[dst_v.at[j]], add=True)
        return carry

    lax.fori_loop(0, NCHUNK, body, 0)
    plsc.subcore_barrier()
    pltpu.sync_copy(
        acc_sh.at[pl.ds(s * RPS, RPS)], out_hbm.at[c, pl.ds(s * RPS, RPS)]
    )


def _prep_body(x_ref, d0_ref, d1_ref, xs_ref, dinv_ref):
    deg = d0_ref[...] + d1_ref[...] + 1.0
    dinv = lax.rsqrt(jnp.maximum(deg, 1e-12))
    dinv_ref[...] = dinv
    xs_ref[...] = x_ref[...] * dinv


_prep_call = pl.pallas_call(
    _prep_body,
    grid=(N // RB,),
    in_specs=[
        pl.BlockSpec((RB, D), lambda i: (i, 0)),
        pl.BlockSpec((RB, 1), lambda i: (i, 0)),
        pl.BlockSpec((RB, 1), lambda i: (i, 0)),
    ],
    out_specs=[
        pl.BlockSpec((RB, D), lambda i: (i, 0)),
        pl.BlockSpec((RB, 1), lambda i: (i, 0)),
    ],
    out_shape=[
        jax.ShapeDtypeStruct((N, D), jnp.float32),
        jax.ShapeDtypeStruct((N, 1), jnp.float32),
    ],
)


def _dense_body(g1a_ref, g1b_ref, xs_ref, dinv_ref, w1_ref, b1_ref, w2_ref,
                zs_ref):
    dinv = dinv_ref[...]
    agg = (g1a_ref[...] + g1b_ref[...] + xs_ref[...]) * dinv
    h = jnp.dot(agg, w1_ref[...], preferred_element_type=jnp.float32)
    h = jnp.maximum(h + b1_ref[...], 0.0)
    z = jnp.dot(h, w2_ref[...], preferred_element_type=jnp.float32)
    zs_ref[...] = z * dinv


_dense_call = pl.pallas_call(
    _dense_body,
    grid=(N // RB,),
    in_specs=[
        pl.BlockSpec((RB, D), lambda i: (i, 0)),
        pl.BlockSpec((RB, D), lambda i: (i, 0)),
        pl.BlockSpec((RB, D), lambda i: (i, 0)),
        pl.BlockSpec((RB, 1), lambda i: (i, 0)),
        pl.BlockSpec((D, 2 * D), lambda i: (0, 0)),
        pl.BlockSpec((1, 2 * D), lambda i: (0, 0)),
        pl.BlockSpec((2 * D, D), lambda i: (0, 0)),
    ],
    out_specs=pl.BlockSpec((RB, D), lambda i: (i, 0)),
    out_shape=jax.ShapeDtypeStruct((N, D), jnp.float32),
)


def _final_body(g2a_ref, g2b_ref, zs_ref, dinv_ref, b2_ref, out_ref):
    out = (g2a_ref[...] + g2b_ref[...] + zs_ref[...]) * dinv_ref[...]
    out_ref[...] = out + b2_ref[...]


_final_call = pl.pallas_call(
    _final_body,
    grid=(N // RB,),
    in_specs=[
        pl.BlockSpec((RB, D), lambda i: (i, 0)),
        pl.BlockSpec((RB, D), lambda i: (i, 0)),
        pl.BlockSpec((RB, D), lambda i: (i, 0)),
        pl.BlockSpec((RB, 1), lambda i: (i, 0)),
        pl.BlockSpec((1, D), lambda i: (0, 0)),
    ],
    out_specs=pl.BlockSpec((RB, D), lambda i: (i, 0)),
    out_shape=jax.ShapeDtypeStruct((N, D), jnp.float32),
)


def kernel(x, edge_index, W1, b1, W2, b2):
    src = edge_index[0].astype(jnp.int32)
    dst = edge_index[1].astype(jnp.int32)
    pi = jnp.arange(E_PAD - E, dtype=jnp.int32)
    src_p = jnp.concatenate([src, pi % N]).reshape(NW, NCHUNK, CHUNK)
    dst_p = jnp.concatenate([dst, N + pi % PAD_ROWS]).reshape(NW, NCHUNK, CHUNK)

    degp = _deg_kernel(dst_p)
    deg0 = degp[:N].reshape(N, 1)
    deg1 = degp[ACC_ROWS:ACC_ROWS + N].reshape(N, 1)

    xs, dinv = _prep_call(x, deg0, deg1)

    g1 = _agg_kernel(xs, src_p, dst_p)
    zs = _dense_call(g1[0, :N], g1[1, :N], xs, dinv,
                     W1, b1.reshape(1, -1), W2)

    g2 = _agg_kernel(zs, src_p, dst_p)
    out = _final_call(g2[0, :N], g2[1, :N], zs, dinv, b2.reshape(1, -1))
    return out


# speedup vs baseline: 34.9285x; 1.4025x over previous
import functools

import jax
import jax.numpy as jnp
from jax import lax
from jax.experimental import pallas as pl
from jax.experimental.pallas import tpu as pltpu
from jax.experimental.pallas import tpu_sc as plsc

N = 10000
D = 128
E = 320000

NC = 2
NS = 16
NW = NC * NS

CHUNK = 128
NCHUNK = 80
BLK = 20
NBLK = NCHUNK // BLK
EPW = NCHUNK * CHUNK
E_PAD = EPW * NW

ACC_ROWS = 10240
PAD_ROWS = ACC_ROWS - N
RPS = ACC_ROWS // NS

RB = 2000


_mesh = plsc.VectorSubcoreMesh(
    core_axis_name="c", subcore_axis_name="s", num_cores=NC, num_subcores=NS
)


@functools.partial(
    pl.kernel,
    out_type=jax.ShapeDtypeStruct((NC * ACC_ROWS,), jnp.float32),
    mesh=_mesh,
    scratch_types=[
        pltpu.VMEM((NCHUNK, CHUNK), jnp.int32),
        pltpu.VMEM((CHUNK,), jnp.float32),
        pltpu.VMEM((RPS,), jnp.float32),
        pltpu.VMEM_SHARED((ACC_ROWS,), jnp.float32),
    ],
)
def _deg_kernel(dst_hbm, out_hbm, dst_v, ones_v, zbuf, acc_sh):
    c = lax.axis_index("c")
    s = lax.axis_index("s")
    w = c * NS + s
    for k in range(CHUNK // 16):
        ones_v[pl.ds(k * 16, 16)] = jnp.ones((16,), jnp.float32)

    def zbody(i, carry):
        zbuf[pl.ds(i * 16, 16)] = jnp.zeros((16,), jnp.float32)
        return carry

    lax.fori_loop(0, RPS // 16, zbody, 0)
    pltpu.sync_copy(zbuf, acc_sh.at[pl.ds(s * RPS, RPS)])
    pltpu.sync_copy(dst_hbm.at[w], dst_v)
    plsc.subcore_barrier()

    def body(j, carry):
        pltpu.sync_copy(ones_v, acc_sh.at[dst_v.at[j]], add=True)
        return carry

    lax.fori_loop(0, NCHUNK, body, 0)
    plsc.subcore_barrier()
    pltpu.sync_copy(acc_sh.at[pl.ds(s * RPS, RPS)], zbuf)
    pltpu.sync_copy(zbuf, out_hbm.at[pl.ds(c * ACC_ROWS + s * RPS, RPS)])


@functools.partial(
    pl.kernel,
    out_type=jax.ShapeDtypeStruct((NC, ACC_ROWS, D), jnp.float32),
    mesh=_mesh,
    scratch_types=[
        pltpu.VMEM((2, BLK, CHUNK), jnp.int32),
        pltpu.VMEM((2, BLK, CHUNK), jnp.int32),
        pltpu.VMEM((2, CHUNK, D), jnp.float32),
        pltpu.VMEM_SHARED((ACC_ROWS, D), jnp.float32),
        pltpu.SemaphoreType.DMA,
        pltpu.SemaphoreType.DMA,
    ],
)
def _agg_kernel(table_hbm, src_hbm, dst_hbm, out_hbm,
                src_v, dst_v, gbuf, acc_sh, sem_g0, sem_g1):
    c = lax.axis_index("c")
    s = lax.axis_index("s")
    w = c * NS + s

    def zbody(i, carry):
        for k in range(D // 16):
            gbuf[0, i, pl.ds(k * 16, 16)] = jnp.zeros((16,), jnp.float32)
        return carry

    lax.fori_loop(0, CHUNK, zbody, 0)

    def zcopy(r, carry):
        pltpu.sync_copy(
            gbuf.at[0], acc_sh.at[pl.ds(s * RPS + r * CHUNK, CHUNK)]
        )
        return carry

    lax.fori_loop(0, RPS // CHUNK, zcopy, 0)
    pltpu.sync_copy(src_hbm.at[w, 0], src_v.at[0])
    pltpu.sync_copy(dst_hbm.at[w, 0], dst_v.at[0])
    plsc.subcore_barrier()

    pltpu.async_copy(table_hbm.at[src_v.at[0, 0]], gbuf.at[0], sem_g0)

    def body(i, carry):
        j0 = 2 * i
        j1 = j0 + 1
        j2 = j0 + 2
        p = (j0 // BLK) % 2
        pltpu.async_copy(
            table_hbm.at[src_v.at[p, j1 % BLK]], gbuf.at[1], sem_g1
        )
        pltpu.make_async_copy(
            table_hbm.at[pl.ds(0, CHUNK)], gbuf.at[0], sem_g0
        ).wait()
        pltpu.sync_copy(gbuf.at[0], acc_sh.at[dst_v.at[p, j0 % BLK]], add=True)

        @pl.when(jnp.logical_and(j2 < NCHUNK, j2 % BLK == 0))
        def _():
            b2 = j2 // BLK
            pltpu.sync_copy(src_hbm.at[w, b2], src_v.at[b2 % 2])
            pltpu.sync_copy(dst_hbm.at[w, b2], dst_v.at[b2 % 2])

        @pl.when(j2 < NCHUNK)
        def _():
            pltpu.async_copy(
                table_hbm.at[src_v.at[(j2 // BLK) % 2, j2 % BLK]],
                gbuf.at[0], sem_g0,
            )

        pltpu.make_async_copy(
            table_hbm.at[pl.ds(0, CHUNK)], gbuf.at[1], sem_g1
        ).wait()
        pltpu.sync_copy(gbuf.at[1], acc_sh.at[dst_v.at[p, j1 % BLK]], add=True)
        return carry

    lax.fori_loop(0, NCHUNK // 2, body, 0)
    plsc.subcore_barrier()
    pltpu.sync_copy(
        acc_sh.at[pl.ds(s * RPS, RPS)], out_hbm.at[c, pl.ds(s * RPS, RPS)]
    )


def _prep_body(x_ref, d0_ref, d1_ref, xs_ref, dinv_ref):
    deg = d0_ref[...] + d1_ref[...] + 1.0
    dinv = lax.rsqrt(jnp.maximum(deg, 1e-12))
    dinv_ref[...] = dinv
    xs_ref[...] = x_ref[...] * dinv


_prep_call = pl.pallas_call(
    _prep_body,
    grid=(N // RB,),
    in_specs=[
        pl.BlockSpec((RB, D), lambda i: (i, 0)),
        pl.BlockSpec((RB, 1), lambda i: (i, 0)),
        pl.BlockSpec((RB, 1), lambda i: (i, 0)),
    ],
    out_specs=[
        pl.BlockSpec((RB, D), lambda i: (i, 0)),
        pl.BlockSpec((RB, 1), lambda i: (i, 0)),
    ],
    out_shape=[
        jax.ShapeDtypeStruct((N, D), jnp.float32),
        jax.ShapeDtypeStruct((N, 1), jnp.float32),
    ],
)


def _dense_body(g1a_ref, g1b_ref, xs_ref, dinv_ref, w1_ref, b1_ref, w2_ref,
                zs_ref):
    dinv = dinv_ref[...]
    agg = (g1a_ref[...] + g1b_ref[...] + xs_ref[...]) * dinv
    h = jnp.dot(agg, w1_ref[...], preferred_element_type=jnp.float32)
    h = jnp.maximum(h + b1_ref[...], 0.0)
    z = jnp.dot(h, w2_ref[...], preferred_element_type=jnp.float32)
    zs_ref[...] = z * dinv


_dense_call = pl.pallas_call(
    _dense_body,
    grid=(N // RB,),
    in_specs=[
        pl.BlockSpec((RB, D), lambda i: (i, 0)),
        pl.BlockSpec((RB, D), lambda i: (i, 0)),
        pl.BlockSpec((RB, D), lambda i: (i, 0)),
        pl.BlockSpec((RB, 1), lambda i: (i, 0)),
        pl.BlockSpec((D, 2 * D), lambda i: (0, 0)),
        pl.BlockSpec((1, 2 * D), lambda i: (0, 0)),
        pl.BlockSpec((2 * D, D), lambda i: (0, 0)),
    ],
    out_specs=pl.BlockSpec((RB, D), lambda i: (i, 0)),
    out_shape=jax.ShapeDtypeStruct((N, D), jnp.float32),
)


def _final_body(g2a_ref, g2b_ref, zs_ref, dinv_ref, b2_ref, out_ref):
    out = (g2a_ref[...] + g2b_ref[...] + zs_ref[...]) * dinv_ref[...]
    out_ref[...] = out + b2_ref[...]


_final_call = pl.pallas_call(
    _final_body,
    grid=(N // RB,),
    in_specs=[
        pl.BlockSpec((RB, D), lambda i: (i, 0)),
        pl.BlockSpec((RB, D), lambda i: (i, 0)),
        pl.BlockSpec((RB, D), lambda i: (i, 0)),
        pl.BlockSpec((RB, 1), lambda i: (i, 0)),
        pl.BlockSpec((1, D), lambda i: (0, 0)),
    ],
    out_specs=pl.BlockSpec((RB, D), lambda i: (i, 0)),
    out_shape=jax.ShapeDtypeStruct((N, D), jnp.float32),
)


def kernel(x, edge_index, W1, b1, W2, b2):
    src = edge_index[0].astype(jnp.int32)
    dst = edge_index[1].astype(jnp.int32)
    pi = jnp.arange(E_PAD - E, dtype=jnp.int32)
    src_p = jnp.concatenate([src, pi % N]).reshape(NW, NCHUNK, CHUNK)
    dst_p = jnp.concatenate([dst, N + pi % PAD_ROWS]).reshape(NW, NCHUNK, CHUNK)

    src_b = src_p.reshape(NW, NBLK, BLK, CHUNK)
    dst_b = dst_p.reshape(NW, NBLK, BLK, CHUNK)

    degp = _deg_kernel(dst_p)
    deg0 = degp[:N].reshape(N, 1)
    deg1 = degp[ACC_ROWS:ACC_ROWS + N].reshape(N, 1)

    xs, dinv = _prep_call(x, deg0, deg1)

    g1 = _agg_kernel(xs, src_b, dst_b)
    zs = _dense_call(g1[0, :N], g1[1, :N], xs, dinv,
                     W1, b1.reshape(1, -1), W2)

    g2 = _agg_kernel(zs, src_b, dst_b)
    out = _final_call(g2[0, :N], g2[1, :N], zs, dinv, b2.reshape(1, -1))
    return out


# speedup vs baseline: 36.5883x; 1.0475x over previous
import functools

import jax
import jax.numpy as jnp
from jax import lax
from jax.experimental import pallas as pl
from jax.experimental.pallas import tpu as pltpu
from jax.experimental.pallas import tpu_sc as plsc

N = 10000
D = 128
E = 320000

NC = 2
NS = 16
NW = NC * NS

CHUNK = 128
NCHUNK = 80
BLK = 20
NBLK = NCHUNK // BLK
EPW = NCHUNK * CHUNK
E_PAD = EPW * NW

ACC_ROWS = 10240
PAD_ROWS = ACC_ROWS - N
RPS = ACC_ROWS // NS

RB = 2000


_mesh = plsc.VectorSubcoreMesh(
    core_axis_name="c", subcore_axis_name="s", num_cores=NC, num_subcores=NS
)


@functools.partial(
    pl.kernel,
    out_type=jax.ShapeDtypeStruct((NC * ACC_ROWS,), jnp.float32),
    mesh=_mesh,
    scratch_types=[
        pltpu.VMEM((NBLK, BLK, CHUNK), jnp.int32),
        pltpu.VMEM((CHUNK,), jnp.float32),
        pltpu.VMEM((RPS,), jnp.float32),
        pltpu.VMEM_SHARED((ACC_ROWS,), jnp.float32),
    ],
)
def _deg_kernel(dst_hbm, out_hbm, dst_v, ones_v, zbuf, acc_sh):
    c = lax.axis_index("c")
    s = lax.axis_index("s")
    w = c * NS + s
    for k in range(CHUNK // 16):
        ones_v[pl.ds(k * 16, 16)] = jnp.ones((16,), jnp.float32)

    def zbody(i, carry):
        zbuf[pl.ds(i * 16, 16)] = jnp.zeros((16,), jnp.float32)
        return carry

    lax.fori_loop(0, RPS // 16, zbody, 0)
    pltpu.sync_copy(zbuf, acc_sh.at[pl.ds(s * RPS, RPS)])
    pltpu.sync_copy(dst_hbm.at[w], dst_v)
    plsc.subcore_barrier()

    def body(j, carry):
        pltpu.sync_copy(ones_v, acc_sh.at[dst_v.at[j // BLK, j % BLK]], add=True)
        return carry

    lax.fori_loop(0, NCHUNK, body, 0)
    plsc.subcore_barrier()
    pltpu.sync_copy(acc_sh.at[pl.ds(s * RPS, RPS)], zbuf)
    pltpu.sync_copy(zbuf, out_hbm.at[pl.ds(c * ACC_ROWS + s * RPS, RPS)])


@functools.partial(
    pl.kernel,
    out_type=jax.ShapeDtypeStruct((NC, ACC_ROWS, D), jnp.float32),
    mesh=_mesh,
    scratch_types=[
        pltpu.VMEM((2, BLK, CHUNK), jnp.int32),
        pltpu.VMEM((2, BLK, CHUNK), jnp.int32),
        pltpu.VMEM((2, CHUNK, D), jnp.float32),
        pltpu.VMEM_SHARED((ACC_ROWS, D), jnp.float32),
        pltpu.SemaphoreType.DMA,
        pltpu.SemaphoreType.DMA,
    ],
)
def _agg_kernel(table_hbm, src_hbm, dst_hbm, out_hbm,
                src_v, dst_v, gbuf, acc_sh, sem_g0, sem_g1):
    c = lax.axis_index("c")
    s = lax.axis_index("s")
    w = c * NS + s

    def zbody(i, carry):
        for k in range(D // 16):
            gbuf[0, i, pl.ds(k * 16, 16)] = jnp.zeros((16,), jnp.float32)
        return carry

    lax.fori_loop(0, CHUNK, zbody, 0)

    def zcopy(r, carry):
        pltpu.sync_copy(
            gbuf.at[0], acc_sh.at[pl.ds(s * RPS + r * CHUNK, CHUNK)]
        )
        return carry

    lax.fori_loop(0, RPS // CHUNK, zcopy, 0)
    pltpu.sync_copy(src_hbm.at[w, 0], src_v.at[0])
    pltpu.sync_copy(dst_hbm.at[w, 0], dst_v.at[0])
    plsc.subcore_barrier()

    pltpu.async_copy(table_hbm.at[src_v.at[0, 0]], gbuf.at[0], sem_g0)

    def body(i, carry):
        j0 = 2 * i
        j1 = j0 + 1
        j2 = j0 + 2
        p = (j0 // BLK) % 2
        pltpu.async_copy(
            table_hbm.at[src_v.at[p, j1 % BLK]], gbuf.at[1], sem_g1
        )
        pltpu.make_async_copy(
            table_hbm.at[pl.ds(0, CHUNK)], gbuf.at[0], sem_g0
        ).wait()
        pltpu.sync_copy(gbuf.at[0], acc_sh.at[dst_v.at[p, j0 % BLK]], add=True)

        @pl.when(jnp.logical_and(j2 < NCHUNK, j2 % BLK == 0))
        def _():
            b2 = j2 // BLK
            pltpu.sync_copy(src_hbm.at[w, b2], src_v.at[b2 % 2])
            pltpu.sync_copy(dst_hbm.at[w, b2], dst_v.at[b2 % 2])

        @pl.when(j2 < NCHUNK)
        def _():
            pltpu.async_copy(
                table_hbm.at[src_v.at[(j2 // BLK) % 2, j2 % BLK]],
                gbuf.at[0], sem_g0,
            )

        pltpu.make_async_copy(
            table_hbm.at[pl.ds(0, CHUNK)], gbuf.at[1], sem_g1
        ).wait()
        pltpu.sync_copy(gbuf.at[1], acc_sh.at[dst_v.at[p, j1 % BLK]], add=True)
        return carry

    lax.fori_loop(0, NCHUNK // 2, body, 0)
    plsc.subcore_barrier()
    pltpu.sync_copy(
        acc_sh.at[pl.ds(s * RPS, RPS)], out_hbm.at[c, pl.ds(s * RPS, RPS)]
    )


def _prep_body(x_ref, d0_ref, d1_ref, xs_ref, dinv_ref):
    deg = d0_ref[...] + d1_ref[...] + 1.0
    dinv = lax.rsqrt(jnp.maximum(deg, 1e-12))
    dinv_ref[...] = dinv
    xs_ref[...] = x_ref[...] * dinv


_prep_call = pl.pallas_call(
    _prep_body,
    grid=(N // RB,),
    in_specs=[
        pl.BlockSpec((RB, D), lambda i: (i, 0)),
        pl.BlockSpec((RB, 1), lambda i: (i, 0)),
        pl.BlockSpec((RB, 1), lambda i: (i, 0)),
    ],
    out_specs=[
        pl.BlockSpec((RB, D), lambda i: (i, 0)),
        pl.BlockSpec((RB, 1), lambda i: (i, 0)),
    ],
    out_shape=[
        jax.ShapeDtypeStruct((N, D), jnp.float32),
        jax.ShapeDtypeStruct((N, 1), jnp.float32),
    ],
)


def _dense_body(g1a_ref, g1b_ref, xs_ref, dinv_ref, w1_ref, b1_ref, w2_ref,
                zs_ref):
    dinv = dinv_ref[...]
    agg = (g1a_ref[0] + g1b_ref[0] + xs_ref[...]) * dinv
    h = jnp.dot(agg, w1_ref[...], preferred_element_type=jnp.float32)
    h = jnp.maximum(h + b1_ref[...], 0.0)
    z = jnp.dot(h, w2_ref[...], preferred_element_type=jnp.float32)
    zs_ref[...] = z * dinv


_dense_call = pl.pallas_call(
    _dense_body,
    grid=(N // RB,),
    in_specs=[
        pl.BlockSpec((1, RB, D), lambda i: (0, i, 0)),
        pl.BlockSpec((1, RB, D), lambda i: (1, i, 0)),
        pl.BlockSpec((RB, D), lambda i: (i, 0)),
        pl.BlockSpec((RB, 1), lambda i: (i, 0)),
        pl.BlockSpec((D, 2 * D), lambda i: (0, 0)),
        pl.BlockSpec((1, 2 * D), lambda i: (0, 0)),
        pl.BlockSpec((2 * D, D), lambda i: (0, 0)),
    ],
    out_specs=pl.BlockSpec((RB, D), lambda i: (i, 0)),
    out_shape=jax.ShapeDtypeStruct((N, D), jnp.float32),
)


def _final_body(g2a_ref, g2b_ref, zs_ref, dinv_ref, b2_ref, out_ref):
    out = (g2a_ref[0] + g2b_ref[0] + zs_ref[...]) * dinv_ref[...]
    out_ref[...] = out + b2_ref[...]


_final_call = pl.pallas_call(
    _final_body,
    grid=(N // RB,),
    in_specs=[
        pl.BlockSpec((1, RB, D), lambda i: (0, i, 0)),
        pl.BlockSpec((1, RB, D), lambda i: (1, i, 0)),
        pl.BlockSpec((RB, D), lambda i: (i, 0)),
        pl.BlockSpec((RB, 1), lambda i: (i, 0)),
        pl.BlockSpec((1, D), lambda i: (0, 0)),
    ],
    out_specs=pl.BlockSpec((RB, D), lambda i: (i, 0)),
    out_shape=jax.ShapeDtypeStruct((N, D), jnp.float32),
)


def kernel(x, edge_index, W1, b1, W2, b2):
    src = edge_index[0].astype(jnp.int32)
    dst = edge_index[1].astype(jnp.int32)
    pi = jnp.arange(E_PAD - E, dtype=jnp.int32)
    src_b = jnp.concatenate([src, pi % N]).reshape(NW, NBLK, BLK, CHUNK)
    dst_b = jnp.concatenate([dst, N + pi % PAD_ROWS]).reshape(NW, NBLK, BLK, CHUNK)

    degp = _deg_kernel(dst_b)
    deg0 = degp[:N].reshape(N, 1)
    deg1 = degp[ACC_ROWS:ACC_ROWS + N].reshape(N, 1)

    xs, dinv = _prep_call(x, deg0, deg1)

    g1 = _agg_kernel(xs, src_b, dst_b)
    zs = _dense_call(g1, g1, xs, dinv, W1, b1.reshape(1, -1), W2)

    g2 = _agg_kernel(zs, src_b, dst_b)
    out = _final_call(g2, g2, zs, dinv, b2.reshape(1, -1))
    return out


# speedup vs baseline: 37.1011x; 1.0140x over previous
import functools

import jax
import jax.numpy as jnp
from jax import lax
from jax.experimental import pallas as pl
from jax.experimental.pallas import tpu as pltpu
from jax.experimental.pallas import tpu_sc as plsc

N = 10000
D = 128
E = 320000

NC = 2
NS = 16
NW = NC * NS

CHUNK = 128
NCHUNK = 80
BLK = 16
NBLK = NCHUNK // BLK
EPW = NCHUNK * CHUNK
E_PAD = EPW * NW

ACC_ROWS = 10240
PAD_ROWS = ACC_ROWS - N
RPS = ACC_ROWS // NS

RB = 2000


_mesh = plsc.VectorSubcoreMesh(
    core_axis_name="c", subcore_axis_name="s", num_cores=NC, num_subcores=NS
)


@functools.partial(
    pl.kernel,
    out_type=jax.ShapeDtypeStruct((NC * ACC_ROWS,), jnp.float32),
    mesh=_mesh,
    scratch_types=[
        pltpu.VMEM((NBLK, BLK, CHUNK), jnp.int32),
        pltpu.VMEM((CHUNK,), jnp.float32),
        pltpu.VMEM((RPS,), jnp.float32),
        pltpu.VMEM_SHARED((ACC_ROWS,), jnp.float32),
    ],
)
def _deg_kernel(eb_hbm, out_hbm, dst_v, ones_v, zbuf, acc_sh):
    c = lax.axis_index("c")
    s = lax.axis_index("s")
    w = c * NS + s
    for k in range(CHUNK // 16):
        ones_v[pl.ds(k * 16, 16)] = jnp.ones((16,), jnp.float32)

    def zbody(i, carry):
        zbuf[pl.ds(i * 16, 16)] = jnp.zeros((16,), jnp.float32)
        return carry

    lax.fori_loop(0, RPS // 16, zbody, 0)
    pltpu.sync_copy(zbuf, acc_sh.at[pl.ds(s * RPS, RPS)])
    pltpu.sync_copy(eb_hbm.at[1, w], dst_v)
    plsc.subcore_barrier()

    def body(j, carry):
        pltpu.sync_copy(ones_v, acc_sh.at[dst_v.at[j // BLK, j % BLK]], add=True)
        return carry

    lax.fori_loop(0, NCHUNK, body, 0)
    plsc.subcore_barrier()
    pltpu.sync_copy(acc_sh.at[pl.ds(s * RPS, RPS)], zbuf)
    pltpu.sync_copy(zbuf, out_hbm.at[pl.ds(c * ACC_ROWS + s * RPS, RPS)])


@functools.partial(
    pl.kernel,
    out_type=jax.ShapeDtypeStruct((NC, ACC_ROWS, D), jnp.float32),
    mesh=_mesh,
    scratch_types=[
        pltpu.VMEM((2, BLK, CHUNK), jnp.int32),
        pltpu.VMEM((2, BLK, CHUNK), jnp.int32),
        pltpu.VMEM((2, CHUNK, D), jnp.float32),
        pltpu.VMEM_SHARED((ACC_ROWS, D), jnp.float32),
        pltpu.SemaphoreType.DMA,
        pltpu.SemaphoreType.DMA,
    ],
)
def _agg_kernel(table_hbm, eb_hbm, out_hbm,
                src_v, dst_v, gbuf, acc_sh, sem_g0, sem_g1):
    c = lax.axis_index("c")
    s = lax.axis_index("s")
    w = c * NS + s

    def zbody(i, carry):
        for k in range(D // 16):
            gbuf[0, i, pl.ds(k * 16, 16)] = jnp.zeros((16,), jnp.float32)
        return carry

    lax.fori_loop(0, CHUNK, zbody, 0)

    def zcopy(r, carry):
        pltpu.sync_copy(
            gbuf.at[0], acc_sh.at[pl.ds(s * RPS + r * CHUNK, CHUNK)]
        )
        return carry

    lax.fori_loop(0, RPS // CHUNK, zcopy, 0)
    pltpu.sync_copy(eb_hbm.at[0, w, 0], src_v.at[0])
    pltpu.sync_copy(eb_hbm.at[1, w, 0], dst_v.at[0])
    plsc.subcore_barrier()

    pltpu.async_copy(table_hbm.at[src_v.at[0, 0]], gbuf.at[0], sem_g0)

    def body(i, carry):
        j0 = 2 * i
        j1 = j0 + 1
        j2 = j0 + 2
        p = (j0 // BLK) % 2
        pltpu.async_copy(
            table_hbm.at[src_v.at[p, j1 % BLK]], gbuf.at[1], sem_g1
        )
        pltpu.make_async_copy(
            table_hbm.at[pl.ds(0, CHUNK)], gbuf.at[0], sem_g0
        ).wait()
        pltpu.sync_copy(gbuf.at[0], acc_sh.at[dst_v.at[p, j0 % BLK]], add=True)

        @pl.when(jnp.logical_and(j2 < NCHUNK, j2 % BLK == 0))
        def _():
            b2 = j2 // BLK
            pltpu.sync_copy(eb_hbm.at[0, w, b2], src_v.at[b2 % 2])
            pltpu.sync_copy(eb_hbm.at[1, w, b2], dst_v.at[b2 % 2])

        @pl.when(j2 < NCHUNK)
        def _():
            pltpu.async_copy(
                table_hbm.at[src_v.at[(j2 // BLK) % 2, j2 % BLK]],
                gbuf.at[0], sem_g0,
            )

        pltpu.make_async_copy(
            table_hbm.at[pl.ds(0, CHUNK)], gbuf.at[1], sem_g1
        ).wait()
        pltpu.sync_copy(gbuf.at[1], acc_sh.at[dst_v.at[p, j1 % BLK]], add=True)
        return carry

    lax.fori_loop(0, NCHUNK // 2, body, 0)
    plsc.subcore_barrier()
    pltpu.sync_copy(
        acc_sh.at[pl.ds(s * RPS, RPS)], out_hbm.at[c, pl.ds(s * RPS, RPS)]
    )


def _prep_body(x_ref, d0_ref, d1_ref, xs_ref, dinv_ref):
    deg = d0_ref[...] + d1_ref[...] + 1.0
    dinv = lax.rsqrt(jnp.maximum(deg, 1e-12))
    dinv_ref[...] = dinv
    xs_ref[...] = x_ref[...] * dinv


_prep_call = pl.pallas_call(
    _prep_body,
    grid=(N // RB,),
    in_specs=[
        pl.BlockSpec((RB, D), lambda i: (i, 0)),
        pl.BlockSpec((RB, 1), lambda i: (i, 0)),
        pl.BlockSpec((RB, 1), lambda i: (i, 0)),
    ],
    out_specs=[
        pl.BlockSpec((RB, D), lambda i: (i, 0)),
        pl.BlockSpec((RB, 1), lambda i: (i, 0)),
    ],
    out_shape=[
        jax.ShapeDtypeStruct((N, D), jnp.float32),
        jax.ShapeDtypeStruct((N, 1), jnp.float32),
    ],
)


def _dense_body(g1a_ref, g1b_ref, xs_ref, dinv_ref, w1_ref, b1_ref, w2_ref,
                zs_ref):
    dinv = dinv_ref[...]
    agg = (g1a_ref[0] + g1b_ref[0] + xs_ref[...]) * dinv
    h = jnp.dot(agg, w1_ref[...], preferred_element_type=jnp.float32)
    h = jnp.maximum(h + b1_ref[...], 0.0)
    z = jnp.dot(h, w2_ref[...], preferred_element_type=jnp.float32)
    zs_ref[...] = z * dinv


_dense_call = pl.pallas_call(
    _dense_body,
    grid=(N // RB,),
    in_specs=[
        pl.BlockSpec((1, RB, D), lambda i: (0, i, 0)),
        pl.BlockSpec((1, RB, D), lambda i: (1, i, 0)),
        pl.BlockSpec((RB, D), lambda i: (i, 0)),
        pl.BlockSpec((RB, 1), lambda i: (i, 0)),
        pl.BlockSpec((D, 2 * D), lambda i: (0, 0)),
        pl.BlockSpec((1, 2 * D), lambda i: (0, 0)),
        pl.BlockSpec((2 * D, D), lambda i: (0, 0)),
    ],
    out_specs=pl.BlockSpec((RB, D), lambda i: (i, 0)),
    out_shape=jax.ShapeDtypeStruct((N, D), jnp.float32),
)


def _final_body(g2a_ref, g2b_ref, zs_ref, dinv_ref, b2_ref, out_ref):
    out = (g2a_ref[0] + g2b_ref[0] + zs_ref[...]) * dinv_ref[...]
    out_ref[...] = out + b2_ref[...]


_final_call = pl.pallas_call(
    _final_body,
    grid=(N // RB,),
    in_specs=[
        pl.BlockSpec((1, RB, D), lambda i: (0, i, 0)),
        pl.BlockSpec((1, RB, D), lambda i: (1, i, 0)),
        pl.BlockSpec((RB, D), lambda i: (i, 0)),
        pl.BlockSpec((RB, 1), lambda i: (i, 0)),
        pl.BlockSpec((1, D), lambda i: (0, 0)),
    ],
    out_specs=pl.BlockSpec((RB, D), lambda i: (i, 0)),
    out_shape=jax.ShapeDtypeStruct((N, D), jnp.float32),
)


def kernel(x, edge_index, W1, b1, W2, b2):
    pi = jnp.arange(E_PAD - E, dtype=jnp.int32)
    pad_pair = jnp.stack([pi % N, N + pi % PAD_ROWS])
    eb = jnp.concatenate(
        [edge_index.astype(jnp.int32), pad_pair], axis=1
    ).reshape(2, NW, NBLK, BLK, CHUNK)

    degp = _deg_kernel(eb)
    deg0 = degp[:N].reshape(N, 1)
    deg1 = degp[ACC_ROWS:ACC_ROWS + N].reshape(N, 1)

    xs, dinv = _prep_call(x, deg0, deg1)

    g1 = _agg_kernel(xs, eb)
    zs = _dense_call(g1, g1, xs, dinv, W1, b1.reshape(1, -1), W2)

    g2 = _agg_kernel(zs, eb)
    out = _final_call(g2, g2, zs, dinv, b2.reshape(1, -1))
    return out


# speedup vs baseline: 39.4372x; 1.0630x over previous
import functools

import jax
import jax.numpy as jnp
from jax import lax
from jax.experimental import pallas as pl
from jax.experimental.pallas import tpu as pltpu
from jax.experimental.pallas import tpu_sc as plsc

N = 10000
D = 128
E = 320000

NC = 2
NS = 16
NW = NC * NS

CHUNK = 128
NCHUNK = 80
BLK = 16
NBLK = NCHUNK // BLK
EPW = NCHUNK * CHUNK
E_PAD = EPW * NW

ACC_ROWS = 10240
PAD_ROWS = ACC_ROWS - N
RPS = ACC_ROWS // NS

RB = 2048
NRB = 5


_mesh = plsc.VectorSubcoreMesh(
    core_axis_name="c", subcore_axis_name="s", num_cores=NC, num_subcores=NS
)


@functools.partial(
    pl.kernel,
    out_type=jax.ShapeDtypeStruct((NC * ACC_ROWS,), jnp.float32),
    mesh=_mesh,
    scratch_types=[
        pltpu.VMEM((NBLK, BLK, CHUNK), jnp.int32),
        pltpu.VMEM((CHUNK,), jnp.float32),
        pltpu.VMEM((RPS,), jnp.float32),
        pltpu.VMEM_SHARED((ACC_ROWS,), jnp.float32),
    ],
)
def _deg_kernel(eb_hbm, out_hbm, dst_v, ones_v, zbuf, acc_sh):
    c = lax.axis_index("c")
    s = lax.axis_index("s")
    w = c * NS + s
    for k in range(CHUNK // 16):
        ones_v[pl.ds(k * 16, 16)] = jnp.ones((16,), jnp.float32)

    def zbody(i, carry):
        zbuf[pl.ds(i * 16, 16)] = jnp.zeros((16,), jnp.float32)
        return carry

    lax.fori_loop(0, RPS // 16, zbody, 0)
    pltpu.sync_copy(zbuf, acc_sh.at[pl.ds(s * RPS, RPS)])
    pltpu.sync_copy(eb_hbm.at[1, w], dst_v)
    plsc.subcore_barrier()

    def body(j, carry):
        pltpu.sync_copy(ones_v, acc_sh.at[dst_v.at[j // BLK, j % BLK]], add=True)
        return carry

    lax.fori_loop(0, NCHUNK, body, 0)
    plsc.subcore_barrier()
    pltpu.sync_copy(acc_sh.at[pl.ds(s * RPS, RPS)], zbuf)
    pltpu.sync_copy(zbuf, out_hbm.at[pl.ds(c * ACC_ROWS + s * RPS, RPS)])


@functools.partial(
    pl.kernel,
    out_type=jax.ShapeDtypeStruct((NC, ACC_ROWS, D), jnp.float32),
    mesh=_mesh,
    scratch_types=[
        pltpu.VMEM((2, BLK, CHUNK), jnp.int32),
        pltpu.VMEM((2, BLK, CHUNK), jnp.int32),
        pltpu.VMEM((2, CHUNK, D), jnp.float32),
        pltpu.VMEM_SHARED((ACC_ROWS, D), jnp.float32),
        pltpu.SemaphoreType.DMA,
        pltpu.SemaphoreType.DMA,
    ],
)
def _agg_kernel(table_hbm, eb_hbm, out_hbm,
                src_v, dst_v, gbuf, acc_sh, sem_g0, sem_g1):
    c = lax.axis_index("c")
    s = lax.axis_index("s")
    w = c * NS + s

    def zbody(i, carry):
        for k in range(D // 16):
            gbuf[0, i, pl.ds(k * 16, 16)] = jnp.zeros((16,), jnp.float32)
        return carry

    lax.fori_loop(0, CHUNK, zbody, 0)

    def zcopy(r, carry):
        pltpu.sync_copy(
            gbuf.at[0], acc_sh.at[pl.ds(s * RPS + r * CHUNK, CHUNK)]
        )
        return carry

    lax.fori_loop(0, RPS // CHUNK, zcopy, 0)
    pltpu.sync_copy(eb_hbm.at[0, w, 0], src_v.at[0])
    pltpu.sync_copy(eb_hbm.at[1, w, 0], dst_v.at[0])
    plsc.subcore_barrier()

    pltpu.async_copy(table_hbm.at[src_v.at[0, 0]], gbuf.at[0], sem_g0)

    def body(i, carry):
        j0 = 2 * i
        j1 = j0 + 1
        j2 = j0 + 2
        p = (j0 // BLK) % 2
        pltpu.async_copy(
            table_hbm.at[src_v.at[p, j1 % BLK]], gbuf.at[1], sem_g1
        )
        pltpu.make_async_copy(
            table_hbm.at[pl.ds(0, CHUNK)], gbuf.at[0], sem_g0
        ).wait()
        pltpu.sync_copy(gbuf.at[0], acc_sh.at[dst_v.at[p, j0 % BLK]], add=True)

        @pl.when(jnp.logical_and(j2 < NCHUNK, j2 % BLK == 0))
        def _():
            b2 = j2 // BLK
            pltpu.sync_copy(eb_hbm.at[0, w, b2], src_v.at[b2 % 2])
            pltpu.sync_copy(eb_hbm.at[1, w, b2], dst_v.at[b2 % 2])

        @pl.when(j2 < NCHUNK)
        def _():
            pltpu.async_copy(
                table_hbm.at[src_v.at[(j2 // BLK) % 2, j2 % BLK]],
                gbuf.at[0], sem_g0,
            )

        pltpu.make_async_copy(
            table_hbm.at[pl.ds(0, CHUNK)], gbuf.at[1], sem_g1
        ).wait()
        pltpu.sync_copy(gbuf.at[1], acc_sh.at[dst_v.at[p, j1 % BLK]], add=True)
        return carry

    lax.fori_loop(0, NCHUNK // 2, body, 0)
    plsc.subcore_barrier()
    pltpu.sync_copy(
        acc_sh.at[pl.ds(s * RPS, RPS)], out_hbm.at[c, pl.ds(s * RPS, RPS)]
    )


def _dinv_block(d0, d1):
    deg = d0 + d1 + 1.0
    return lax.rsqrt(jnp.maximum(deg, 1e-12)).reshape(RB, 1)


def _prep_body(x_ref, d0_ref, d1_ref, xs_ref):
    xs_ref[...] = x_ref[...] * _dinv_block(d0_ref[...], d1_ref[...])


_prep_call = pl.pallas_call(
    _prep_body,
    grid=(NRB,),
    in_specs=[
        pl.BlockSpec((RB, D), lambda i: (i, 0)),
        pl.BlockSpec((RB,), lambda i: (i,)),
        pl.BlockSpec((RB,), lambda i: (i + NRB,)),
    ],
    out_specs=pl.BlockSpec((RB, D), lambda i: (i, 0)),
    out_shape=jax.ShapeDtypeStruct((N, D), jnp.float32),
)


def _dense_body(g1a_ref, g1b_ref, xs_ref, d0_ref, d1_ref, w1_ref, b1_ref,
                w2_ref, zs_ref):
    dinv = _dinv_block(d0_ref[...], d1_ref[...])
    agg = (g1a_ref[0] + g1b_ref[0] + xs_ref[...]) * dinv
    h = jnp.dot(agg, w1_ref[...], preferred_element_type=jnp.float32)
    h = jnp.maximum(h + b1_ref[...], 0.0)
    z = jnp.dot(h, w2_ref[...], preferred_element_type=jnp.float32)
    zs_ref[...] = z * dinv


_dense_call = pl.pallas_call(
    _dense_body,
    grid=(NRB,),
    in_specs=[
        pl.BlockSpec((1, RB, D), lambda i: (0, i, 0)),
        pl.BlockSpec((1, RB, D), lambda i: (1, i, 0)),
        pl.BlockSpec((RB, D), lambda i: (i, 0)),
        pl.BlockSpec((RB,), lambda i: (i,)),
        pl.BlockSpec((RB,), lambda i: (i + NRB,)),
        pl.BlockSpec((D, 2 * D), lambda i: (0, 0)),
        pl.BlockSpec((1, 2 * D), lambda i: (0, 0)),
        pl.BlockSpec((2 * D, D), lambda i: (0, 0)),
    ],
    out_specs=pl.BlockSpec((RB, D), lambda i: (i, 0)),
    out_shape=jax.ShapeDtypeStruct((N, D), jnp.float32),
)


def _final_body(g2a_ref, g2b_ref, zs_ref, d0_ref, d1_ref, b2_ref, out_ref):
    dinv = _dinv_block(d0_ref[...], d1_ref[...])
    out = (g2a_ref[0] + g2b_ref[0] + zs_ref[...]) * dinv
    out_ref[...] = out + b2_ref[...]


_final_call = pl.pallas_call(
    _final_body,
    grid=(NRB,),
    in_specs=[
        pl.BlockSpec((1, RB, D), lambda i: (0, i, 0)),
        pl.BlockSpec((1, RB, D), lambda i: (1, i, 0)),
        pl.BlockSpec((RB, D), lambda i: (i, 0)),
        pl.BlockSpec((RB,), lambda i: (i,)),
        pl.BlockSpec((RB,), lambda i: (i + NRB,)),
        pl.BlockSpec((1, D), lambda i: (0, 0)),
    ],
    out_specs=pl.BlockSpec((RB, D), lambda i: (i, 0)),
    out_shape=jax.ShapeDtypeStruct((N, D), jnp.float32),
)


def kernel(x, edge_index, W1, b1, W2, b2):
    pi = jnp.arange(E_PAD - E, dtype=jnp.int32)
    pad_pair = jnp.stack([pi % N, N + pi % PAD_ROWS])
    eb = jnp.concatenate(
        [edge_index.astype(jnp.int32), pad_pair], axis=1
    ).reshape(2, NW, NBLK, BLK, CHUNK)

    degp = _deg_kernel(eb)

    xs = _prep_call(x, degp, degp)

    g1 = _agg_kernel(xs, eb)
    zs = _dense_call(g1, g1, xs, degp, degp, W1, b1.reshape(1, -1), W2)

    g2 = _agg_kernel(zs, eb)
    out = _final_call(g2, g2, zs, degp, degp, b2.reshape(1, -1))
    return out
